# Initial kernel scaffold; baseline (speedup 1.0000x reference)
import jax
import jax.numpy as jnp
from jax.experimental import pallas as pl


def kernel(x, edge_index, edge_attr, batch, emb_W1, emb_b1, emb_W2, emb_b2, msg_W1, msg_b1, msg_W2, msg_b2, upd_W1, upd_b1, upd_W2, upd_b2, head_W1, head_b1, head_W2, head_b2):
    raise NotImplementedError("write your pallas kernel here")


# speedup vs baseline: 1.8251x; 1.8251x over previous
import functools

import jax
import jax.numpy as jnp
from jax import lax
from jax.experimental import pallas as pl
from jax.experimental.pallas import tpu as pltpu
from jax.experimental.pallas import tpu_sc as plsc

N = 10000
NP = 10240
E = 160000
EP = 163840
D = 128
DH = 64
DE = 16
L = 4
G = 64
KC = 128
NCHUNK = EP // KC
CPS = NCHUNK // 16
CPW = NCHUNK // 32
RPT = NP // 16
F32 = jnp.float32


def _dot(a, b):
    return jnp.dot(a, b, preferred_element_type=F32)


def _mlp_body(x_ref, w1, b1, w2, b2, o_ref):
    t = jnp.maximum(_dot(x_ref[...], w1[...]) + b1[...], 0.0)
    o_ref[...] = _dot(t, w2[...]) + b2[...]


def _embed(x, w1, b1, w2, b2):
    blk = NP // 8
    return pl.pallas_call(
        _mlp_body,
        grid=(8,),
        in_specs=[
            pl.BlockSpec((blk, D), lambda i: (i, 0)),
            pl.BlockSpec((D, D), lambda i: (0, 0)),
            pl.BlockSpec((1, D), lambda i: (0, 0)),
            pl.BlockSpec((D, D), lambda i: (0, 0)),
            pl.BlockSpec((1, D), lambda i: (0, 0)),
        ],
        out_specs=pl.BlockSpec((blk, D), lambda i: (i, 0)),
        out_shape=jax.ShapeDtypeStruct((NP, D), F32),
    )(x, w1, b1.reshape(1, D), w2, b2.reshape(1, D))


def _cmat_body(ea_ref, w_ref, b_ref, o_ref):
    r = _dot(ea_ref[...], w_ref[0]) + b_ref[0]
    o_ref[0, 0] = r[:, :DH]
    o_ref[0, 1] = r[:, DH:]


def _cmat(ea_pad, w_e, b1):
    blk = 2048
    return pl.pallas_call(
        _cmat_body,
        grid=(L, EP // blk),
        in_specs=[
            pl.BlockSpec((blk, DE), lambda l, i: (i, 0)),
            pl.BlockSpec((1, DE, D), lambda l, i: (l, 0, 0)),
            pl.BlockSpec((1, 1, D), lambda l, i: (l, 0, 0)),
        ],
        out_specs=pl.BlockSpec((1, 2, blk, DH), lambda l, i: (l, 0, i, 0)),
        out_shape=jax.ShapeDtypeStruct((L, 2, EP, DH), F32),
    )(ea_pad, w_e, b1.reshape(L, 1, D))


def _ab_body(h_ref, wa, wb, a_ref, b_ref):
    h = h_ref[...]
    a = _dot(h, wa[...])
    b = _dot(h, wb[...])
    a_ref[0] = a[:, :DH]
    a_ref[1] = a[:, DH:]
    b_ref[0] = b[:, :DH]
    b_ref[1] = b[:, DH:]


def _ab(h, wa, wb):
    blk = NP // 8
    return pl.pallas_call(
        _ab_body,
        grid=(8,),
        in_specs=[
            pl.BlockSpec((blk, D), lambda i: (i, 0)),
            pl.BlockSpec((D, D), lambda i: (0, 0)),
            pl.BlockSpec((D, D), lambda i: (0, 0)),
        ],
        out_specs=[
            pl.BlockSpec((2, blk, DH), lambda i: (0, i, 0)),
            pl.BlockSpec((2, blk, DH), lambda i: (0, i, 0)),
        ],
        out_shape=[
            jax.ShapeDtypeStruct((2, NP, DH), F32),
            jax.ShapeDtypeStruct((2, NP, DH), F32),
        ],
    )(h, wa, wb)


def _upd_body(h_ref, p_ref, d_ref, w2, b2, u1h, u1a, u2, ub1, ub2, o_ref):
    s = jnp.concatenate([p_ref[0], p_ref[1]], axis=-1)
    deg = d_ref[0, :, 0:1] + d_ref[1, :, 0:1]
    agg = _dot(s, w2[...]) + deg * b2[...]
    pre = _dot(h_ref[...], u1h[...]) + _dot(agg, u1a[...]) + ub1[...]
    o_ref[...] = _dot(jnp.maximum(pre, 0.0), u2[...]) + ub2[...]


def _update(h, parts, degp, w2, b2, u1h, u1a, u2, ub1, ub2):
    blk = NP // 8
    return pl.pallas_call(
        _upd_body,
        grid=(8,),
        in_specs=[
            pl.BlockSpec((blk, D), lambda i: (i, 0)),
            pl.BlockSpec((2, blk, DH), lambda i: (0, i, 0)),
            pl.BlockSpec((2, blk, DE), lambda i: (0, i, 0)),
            pl.BlockSpec((D, D), lambda i: (0, 0)),
            pl.BlockSpec((1, D), lambda i: (0, 0)),
            pl.BlockSpec((D, D), lambda i: (0, 0)),
            pl.BlockSpec((D, D), lambda i: (0, 0)),
            pl.BlockSpec((D, D), lambda i: (0, 0)),
            pl.BlockSpec((1, D), lambda i: (0, 0)),
            pl.BlockSpec((1, D), lambda i: (0, 0)),
        ],
        out_specs=pl.BlockSpec((blk, D), lambda i: (i, 0)),
        out_shape=jax.ShapeDtypeStruct((NP, D), F32),
    )(h, parts, degp, w2, b2.reshape(1, D), u1h, u1a, u2,
      ub1.reshape(1, D), ub2.reshape(1, D))


def _pool_body(h_ref, b_ref, hw1, hb1, hw2, hb2, o_ref, sums, cnts):
    i = pl.program_id(0)

    @pl.when(i == 0)
    def _():
        sums[...] = jnp.zeros_like(sums)
        cnts[...] = jnp.zeros_like(cnts)

    col = b_ref[0].reshape(-1, 1)
    gid = lax.broadcasted_iota(jnp.int32, (col.shape[0], G), 1).astype(F32)
    onehot = jnp.where(col == gid, 1.0, 0.0)
    sums[...] += lax.dot_general(onehot, h_ref[...],
                                 (((0,), (0,)), ((), ())),
                                 preferred_element_type=F32)
    cnts[...] += jnp.sum(onehot, axis=0)[:, None]

    @pl.when(i == pl.num_programs(0) - 1)
    def _():
        pooled = sums[...] / jnp.maximum(cnts[...], 1.0)
        t = jnp.maximum(_dot(pooled, hw1[...]) + hb1[...], 0.0)
        o_ref[...] = _dot(t, hw2[...]) + hb2[...]


def _pool_head(h, batch_f, hw1, hb1, hw2, hb2):
    blk = 1000
    return pl.pallas_call(
        _pool_body,
        grid=(N // blk,),
        in_specs=[
            pl.BlockSpec((blk, D), lambda i: (i, 0)),
            pl.BlockSpec((1, 1, blk), lambda i: (i, 0, 0)),
            pl.BlockSpec((D, D), lambda i: (0, 0)),
            pl.BlockSpec((1, D), lambda i: (0, 0)),
            pl.BlockSpec((D, D), lambda i: (0, 0)),
            pl.BlockSpec((1, D), lambda i: (0, 0)),
        ],
        out_specs=pl.BlockSpec((G, D), lambda i: (0, 0)),
        out_shape=jax.ShapeDtypeStruct((G, D), F32),
        scratch_shapes=[
            pltpu.VMEM((G, D), F32),
            pltpu.VMEM((G, D), F32),
        ],
    )(h, batch_f, hw1, hb1.reshape(1, D), hw2, hb2.reshape(1, D))


_SC_MESH = dict(core_axis_name="c", subcore_axis_name="s")


def _make_deg_kernel():
    mesh = plsc.VectorSubcoreMesh(**_SC_MESH)
    scratch = [
        pltpu.VMEM((CPW, KC), jnp.int32),
        pltpu.VMEM((KC, DE), F32),
        pltpu.VMEM_SHARED((NP, DE), F32),
    ]

    def fn(dst_hbm, zd_hbm, ones_hbm, degp_hbm, dsti, ones_v, acc):
        c = lax.axis_index("c")
        s = lax.axis_index("s")
        wid = s * 2 + c
        pltpu.sync_copy(dst_hbm.at[pl.ds(wid * CPW, CPW)], dsti)
        rows = pl.ds(s * RPT, RPT)
        pltpu.sync_copy(zd_hbm.at[rows], acc.at[rows])
        pltpu.sync_copy(ones_hbm, ones_v)
        plsc.subcore_barrier()

        def chunk(j, carry):
            pltpu.sync_copy(ones_v, acc.at[dsti.at[j]], add=True)
            return carry

        lax.fori_loop(0, CPW, chunk, 0)
        plsc.subcore_barrier()
        pltpu.sync_copy(acc.at[rows], degp_hbm.at[c, rows])

    return functools.partial(
        pl.kernel, mesh=mesh,
        out_type=[jax.ShapeDtypeStruct((2, NP, DE), F32)],
        compiler_params=pltpu.CompilerParams(use_tc_tiling_on_sc=False),
        scratch_types=scratch)(fn)


def _make_edge_kernel(lidx):
    mesh = plsc.VectorSubcoreMesh(**_SC_MESH)
    scratch = [
        pltpu.VMEM((CPS, KC), jnp.int32),
        pltpu.VMEM((CPS, KC), jnp.int32),
        pltpu.VMEM((KC, DH), F32),
        pltpu.VMEM((KC, DH), F32),
        pltpu.VMEM((KC, DH), F32),
        pltpu.VMEM_SHARED((NP, DH), F32),
        pltpu.SemaphoreType.DMA,
    ]

    def fn(a_hbm, b_hbm, c_hbm, dst_hbm, src_hbm, zn_hbm,
           parts_hbm, dsti, srci, arow, brow, crow, acc, sem):
        c = lax.axis_index("c")
        s = lax.axis_index("s")
        pltpu.sync_copy(dst_hbm.at[pl.ds(s * CPS, CPS)], dsti)
        pltpu.sync_copy(src_hbm.at[pl.ds(s * CPS, CPS)], srci)
        rows = pl.ds(s * RPT, RPT)
        pltpu.sync_copy(zn_hbm.at[rows], acc.at[rows])
        plsc.subcore_barrier()

        def run(hf):
            def chunk(j, carry):
                base = s * CPS + j
                asrc = a_hbm.at[hf].at[dsti.at[j]]
                bsrc = b_hbm.at[hf].at[srci.at[j]]
                csrc = c_hbm.at[lidx, hf, pl.ds(base * KC, KC)]
                pltpu.async_copy(asrc, arow, sem)
                pltpu.async_copy(bsrc, brow, sem)
                pltpu.async_copy(csrc, crow, sem)
                pltpu.make_async_copy(asrc, arow, sem).wait()
                pltpu.make_async_copy(bsrc, brow, sem).wait()
                pltpu.make_async_copy(csrc, crow, sem).wait()

                def row(r, carry2):
                    for q in range(DH // 16):
                        sl = pl.ds(q * 16, 16)
                        v = arow[r, sl] + brow[r, sl] + crow[r, sl]
                        arow[r, sl] = jnp.maximum(v, 0.0)
                    return carry2

                lax.fori_loop(0, KC, row, 0)
                pltpu.sync_copy(arow, acc.at[dsti.at[j]], add=True)
                return carry

            lax.fori_loop(0, CPS, chunk, 0)

        @pl.when(c == 0)
        def _():
            run(0)

        @pl.when(c == 1)
        def _():
            run(1)

        plsc.subcore_barrier()
        pltpu.sync_copy(acc.at[rows], parts_hbm.at[c, rows])

    return functools.partial(
        pl.kernel, mesh=mesh,
        out_type=[jax.ShapeDtypeStruct((2, NP, DH), F32)],
        compiler_params=pltpu.CompilerParams(use_tc_tiling_on_sc=False),
        scratch_types=scratch)(fn)


def kernel(x, edge_index, edge_attr, batch,
           emb_W1, emb_b1, emb_W2, emb_b2,
           msg_W1, msg_b1, msg_W2, msg_b2,
           upd_W1, upd_b1, upd_W2, upd_b2,
           head_W1, head_b1, head_W2, head_b2):
    src = edge_index[0]
    dst = edge_index[1]
    x_pad = jnp.pad(x, ((0, NP - N), (0, 0)))
    dst2 = jnp.pad(dst, (0, EP - E), constant_values=N).reshape(NCHUNK, KC)
    src2 = jnp.pad(src, (0, EP - E)).reshape(NCHUNK, KC)
    ea_pad = jnp.pad(edge_attr, ((0, EP - E), (0, 0)))
    batch_f = batch.astype(F32).reshape(N // 1000, 1, 1000)
    zn = jnp.zeros((NP, DH), F32)
    zd = jnp.zeros((NP, DE), F32)
    ones_rows = jnp.ones((KC, DE), F32)

    h = _embed(x_pad, emb_W1, emb_b1, emb_W2, emb_b2)
    c_all = _cmat(ea_pad, msg_W1[:, 2 * D:, :], msg_b1)
    (degp,) = _make_deg_kernel()(dst2, zd, ones_rows)

    for l in range(L):
        a_t, b_t = _ab(h, msg_W1[l, :D], msg_W1[l, D:2 * D])
        (parts,) = _make_edge_kernel(l)(a_t, b_t, c_all, dst2, src2, zn)
        h = _update(h, parts, degp, msg_W2[l], msg_b2[l],
                    upd_W1[l, :D], upd_W1[l, D:], upd_W2[l],
                    upd_b1[l], upd_b2[l])

    return _pool_head(h, batch_f, head_W1, head_b1, head_W2, head_b2)


# speedup vs baseline: 2.2134x; 1.2127x over previous
import functools

import jax
import jax.numpy as jnp
from jax import lax
from jax.experimental import pallas as pl
from jax.experimental.pallas import tpu as pltpu
from jax.experimental.pallas import tpu_sc as plsc

N = 10000
NP = 10240
E = 160000
EP = 163840
D = 128
DH = 64
DE = 16
L = 4
G = 64
KC = 128
NCHUNK = EP // KC
CPS = NCHUNK // 16
CPW = NCHUNK // 32
RPT = NP // 16
F32 = jnp.float32


def _dot(a, b):
    return jnp.dot(a, b, preferred_element_type=F32)


def _mlp_body(x_ref, w1, b1, w2, b2, o_ref):
    t = jnp.maximum(_dot(x_ref[...], w1[...]) + b1[...], 0.0)
    o_ref[...] = _dot(t, w2[...]) + b2[...]


def _embed(x, w1, b1, w2, b2):
    blk = NP // 8
    return pl.pallas_call(
        _mlp_body,
        grid=(8,),
        in_specs=[
            pl.BlockSpec((blk, D), lambda i: (i, 0)),
            pl.BlockSpec((D, D), lambda i: (0, 0)),
            pl.BlockSpec((1, D), lambda i: (0, 0)),
            pl.BlockSpec((D, D), lambda i: (0, 0)),
            pl.BlockSpec((1, D), lambda i: (0, 0)),
        ],
        out_specs=pl.BlockSpec((blk, D), lambda i: (i, 0)),
        out_shape=jax.ShapeDtypeStruct((NP, D), F32),
    )(x, w1, b1.reshape(1, D), w2, b2.reshape(1, D))


def _cmat_body(ea_ref, w_ref, b_ref, o_ref):
    r = _dot(ea_ref[...], w_ref[0]) + b_ref[0]
    o_ref[0, 0] = r[:, :DH]
    o_ref[0, 1] = r[:, DH:]


def _cmat(ea_pad, w_e, b1):
    blk = 2048
    return pl.pallas_call(
        _cmat_body,
        grid=(L, EP // blk),
        in_specs=[
            pl.BlockSpec((blk, DE), lambda l, i: (i, 0)),
            pl.BlockSpec((1, DE, D), lambda l, i: (l, 0, 0)),
            pl.BlockSpec((1, 1, D), lambda l, i: (l, 0, 0)),
        ],
        out_specs=pl.BlockSpec((1, 2, blk, DH), lambda l, i: (l, 0, i, 0)),
        out_shape=jax.ShapeDtypeStruct((L, 2, EP, DH), F32),
    )(ea_pad, w_e, b1.reshape(L, 1, D))


def _ab_body(h_ref, wa, wb, a_ref, b_ref):
    h = h_ref[...]
    a = _dot(h, wa[...])
    b = _dot(h, wb[...])
    a_ref[0] = a[:, :DH]
    a_ref[1] = a[:, DH:]
    b_ref[0] = b[:, :DH]
    b_ref[1] = b[:, DH:]


def _ab(h, wa, wb):
    blk = NP // 8
    return pl.pallas_call(
        _ab_body,
        grid=(8,),
        in_specs=[
            pl.BlockSpec((blk, D), lambda i: (i, 0)),
            pl.BlockSpec((D, D), lambda i: (0, 0)),
            pl.BlockSpec((D, D), lambda i: (0, 0)),
        ],
        out_specs=[
            pl.BlockSpec((2, blk, DH), lambda i: (0, i, 0)),
            pl.BlockSpec((2, blk, DH), lambda i: (0, i, 0)),
        ],
        out_shape=[
            jax.ShapeDtypeStruct((2, NP, DH), F32),
            jax.ShapeDtypeStruct((2, NP, DH), F32),
        ],
    )(h, wa, wb)


def _upd_body(h_ref, p0_ref, p1_ref, d0_ref, d1_ref,
              w2, b2, u1h, u1a, u2, ub1, ub2, o_ref):
    s = jnp.concatenate([p0_ref[...], p1_ref[...]], axis=-1)
    deg = d0_ref[:, 0:1] + d1_ref[:, 0:1]
    agg = _dot(s, w2[...]) + deg * b2[...]
    pre = _dot(h_ref[...], u1h[...]) + _dot(agg, u1a[...]) + ub1[...]
    o_ref[...] = _dot(jnp.maximum(pre, 0.0), u2[...]) + ub2[...]


def _update(h, p0, p1, d0, d1, w2, b2, u1h, u1a, u2, ub1, ub2):
    blk = NP // 8
    return pl.pallas_call(
        _upd_body,
        grid=(8,),
        in_specs=[
            pl.BlockSpec((blk, D), lambda i: (i, 0)),
            pl.BlockSpec((blk, DH), lambda i: (i, 0)),
            pl.BlockSpec((blk, DH), lambda i: (i, 0)),
            pl.BlockSpec((blk, DE), lambda i: (i, 0)),
            pl.BlockSpec((blk, DE), lambda i: (i, 0)),
            pl.BlockSpec((D, D), lambda i: (0, 0)),
            pl.BlockSpec((1, D), lambda i: (0, 0)),
            pl.BlockSpec((D, D), lambda i: (0, 0)),
            pl.BlockSpec((D, D), lambda i: (0, 0)),
            pl.BlockSpec((D, D), lambda i: (0, 0)),
            pl.BlockSpec((1, D), lambda i: (0, 0)),
            pl.BlockSpec((1, D), lambda i: (0, 0)),
        ],
        out_specs=pl.BlockSpec((blk, D), lambda i: (i, 0)),
        out_shape=jax.ShapeDtypeStruct((NP, D), F32),
    )(h, p0, p1, d0, d1, w2, b2.reshape(1, D), u1h, u1a, u2,
      ub1.reshape(1, D), ub2.reshape(1, D))


def _pool_body(h_ref, b_ref, hw1, hb1, hw2, hb2, o_ref, sums, cnts):
    i = pl.program_id(0)

    @pl.when(i == 0)
    def _():
        sums[...] = jnp.zeros_like(sums)
        cnts[...] = jnp.zeros_like(cnts)

    col = b_ref[0].reshape(-1, 1)
    gid = lax.broadcasted_iota(jnp.int32, (col.shape[0], G), 1).astype(F32)
    onehot = jnp.where(col == gid, 1.0, 0.0)
    sums[...] += lax.dot_general(onehot, h_ref[...],
                                 (((0,), (0,)), ((), ())),
                                 preferred_element_type=F32)
    cnts[...] += jnp.sum(onehot, axis=0)[:, None]

    @pl.when(i == pl.num_programs(0) - 1)
    def _():
        pooled = sums[...] / jnp.maximum(cnts[...], 1.0)
        t = jnp.maximum(_dot(pooled, hw1[...]) + hb1[...], 0.0)
        o_ref[...] = _dot(t, hw2[...]) + hb2[...]


def _pool_head(h, batch_f, hw1, hb1, hw2, hb2):
    blk = 1000
    return pl.pallas_call(
        _pool_body,
        grid=(N // blk,),
        in_specs=[
            pl.BlockSpec((blk, D), lambda i: (i, 0)),
            pl.BlockSpec((1, 1, blk), lambda i: (i, 0, 0)),
            pl.BlockSpec((D, D), lambda i: (0, 0)),
            pl.BlockSpec((1, D), lambda i: (0, 0)),
            pl.BlockSpec((D, D), lambda i: (0, 0)),
            pl.BlockSpec((1, D), lambda i: (0, 0)),
        ],
        out_specs=pl.BlockSpec((G, D), lambda i: (0, 0)),
        out_shape=jax.ShapeDtypeStruct((G, D), F32),
        scratch_shapes=[
            pltpu.VMEM((G, D), F32),
            pltpu.VMEM((G, D), F32),
        ],
    )(h, batch_f, hw1, hb1.reshape(1, D), hw2, hb2.reshape(1, D))


_SC_MESH = dict(core_axis_name="c", subcore_axis_name="s")


def _make_deg_kernel():
    mesh = plsc.VectorSubcoreMesh(**_SC_MESH)
    scratch = [
        pltpu.VMEM((CPW, KC), jnp.int32),
        pltpu.VMEM((KC, DE), F32),
        pltpu.VMEM_SHARED((NP, DE), F32),
    ]

    def fn(dst_hbm, zd_hbm, ones_hbm, deg0_hbm, deg1_hbm, dsti, ones_v, acc):
        c = lax.axis_index("c")
        s = lax.axis_index("s")
        wid = s * 2 + c
        pltpu.sync_copy(dst_hbm.at[pl.ds(wid * CPW, CPW)], dsti)
        rows = pl.ds(s * RPT, RPT)
        pltpu.sync_copy(zd_hbm.at[rows], acc.at[rows])
        pltpu.sync_copy(ones_hbm, ones_v)
        plsc.subcore_barrier()

        def chunk(j, carry):
            pltpu.sync_copy(ones_v, acc.at[dsti.at[j]], add=True)
            return carry

        lax.fori_loop(0, CPW, chunk, 0)
        plsc.subcore_barrier()

        @pl.when(c == 0)
        def _():
            pltpu.sync_copy(acc.at[rows], deg0_hbm.at[rows])

        @pl.when(c == 1)
        def _():
            pltpu.sync_copy(acc.at[rows], deg1_hbm.at[rows])

    return functools.partial(
        pl.kernel, mesh=mesh,
        out_type=[jax.ShapeDtypeStruct((NP, DE), F32),
                  jax.ShapeDtypeStruct((NP, DE), F32)],
        compiler_params=pltpu.CompilerParams(use_tc_tiling_on_sc=False),
        scratch_types=scratch)(fn)


def _make_edge_kernel(lidx):
    mesh = plsc.VectorSubcoreMesh(**_SC_MESH)
    scratch = [
        pltpu.VMEM((CPS, KC), jnp.int32),
        pltpu.VMEM((CPS, KC), jnp.int32),
        pltpu.VMEM((KC, DH), F32),
        pltpu.VMEM((KC, DH), F32),
        pltpu.VMEM((KC, DH), F32),
        pltpu.VMEM((KC, DH), F32),
        pltpu.VMEM((KC, DH), F32),
        pltpu.VMEM((KC, DH), F32),
        pltpu.VMEM_SHARED((NP, DH), F32),
        pltpu.SemaphoreType.DMA,
        pltpu.SemaphoreType.DMA,
    ]

    def fn(a_hbm, b_hbm, c_hbm, dst_hbm, src_hbm, zn_hbm,
           p0_hbm, p1_hbm, dsti, srci,
           ar0, br0, cr0, ar1, br1, cr1, acc, sem0, sem1):
        c = lax.axis_index("c")
        s = lax.axis_index("s")
        pltpu.sync_copy(dst_hbm.at[pl.ds(s * CPS, CPS)], dsti)
        pltpu.sync_copy(src_hbm.at[pl.ds(s * CPS, CPS)], srci)
        rows = pl.ds(s * RPT, RPT)
        pltpu.sync_copy(zn_hbm.at[rows], acc.at[rows])
        plsc.subcore_barrier()

        def run(hf):
            def copies(j, ar, br, cr, sem):
                base = s * CPS + j
                return (
                    pltpu.make_async_copy(a_hbm.at[hf].at[dsti.at[j]],
                                          ar, sem),
                    pltpu.make_async_copy(b_hbm.at[hf].at[srci.at[j]],
                                          br, sem),
                    pltpu.make_async_copy(
                        c_hbm.at[lidx, hf, pl.ds(base * KC, KC)], cr, sem),
                )

            def issue(j, ar, br, cr, sem):
                for cp in copies(j, ar, br, cr, sem):
                    cp.start()

            def drain(j, ar, br, cr, sem):
                for cp in copies(j, ar, br, cr, sem):
                    cp.wait()

            def compute_scatter(j, ar, br, cr):
                def row(r, carry2):
                    for q in range(DH // 16):
                        sl = pl.ds(q * 16, 16)
                        v = ar[r, sl] + br[r, sl] + cr[r, sl]
                        ar[r, sl] = jnp.maximum(v, 0.0)
                    return carry2

                lax.fori_loop(0, KC, row, 0)
                pltpu.sync_copy(ar, acc.at[dsti.at[j]], add=True)

            issue(0, ar0, br0, cr0, sem0)

            def pair(jj, carry):
                j0 = 2 * jj
                j1 = j0 + 1
                issue(j1, ar1, br1, cr1, sem1)
                drain(j0, ar0, br0, cr0, sem0)
                compute_scatter(j0, ar0, br0, cr0)

                @pl.when(jj < CPS // 2 - 1)
                def _():
                    issue(j0 + 2, ar0, br0, cr0, sem0)

                drain(j1, ar1, br1, cr1, sem1)
                compute_scatter(j1, ar1, br1, cr1)
                return carry

            lax.fori_loop(0, CPS // 2, pair, 0)

        @pl.when(c == 0)
        def _():
            run(0)

        @pl.when(c == 1)
        def _():
            run(1)

        plsc.subcore_barrier()

        @pl.when(c == 0)
        def _():
            pltpu.sync_copy(acc.at[rows], p0_hbm.at[rows])

        @pl.when(c == 1)
        def _():
            pltpu.sync_copy(acc.at[rows], p1_hbm.at[rows])

    return functools.partial(
        pl.kernel, mesh=mesh,
        out_type=[jax.ShapeDtypeStruct((NP, DH), F32),
                  jax.ShapeDtypeStruct((NP, DH), F32)],
        compiler_params=pltpu.CompilerParams(use_tc_tiling_on_sc=False),
        scratch_types=scratch)(fn)


def kernel(x, edge_index, edge_attr, batch,
           emb_W1, emb_b1, emb_W2, emb_b2,
           msg_W1, msg_b1, msg_W2, msg_b2,
           upd_W1, upd_b1, upd_W2, upd_b2,
           head_W1, head_b1, head_W2, head_b2):
    src = edge_index[0]
    dst = edge_index[1]
    x_pad = jnp.pad(x, ((0, NP - N), (0, 0)))
    dst2 = jnp.pad(dst, (0, EP - E), constant_values=N).reshape(NCHUNK, KC)
    src2 = jnp.pad(src, (0, EP - E)).reshape(NCHUNK, KC)
    ea_pad = jnp.pad(edge_attr, ((0, EP - E), (0, 0)))
    batch_f = batch.astype(F32).reshape(N // 1000, 1, 1000)
    zn = jnp.zeros((NP, DH), F32)
    zd = jnp.zeros((NP, DE), F32)
    ones_rows = jnp.ones((KC, DE), F32)

    h = _embed(x_pad, emb_W1, emb_b1, emb_W2, emb_b2)
    c_all = _cmat(ea_pad, msg_W1[:, 2 * D:, :], msg_b1)
    d0, d1 = _make_deg_kernel()(dst2, zd, ones_rows)

    for l in range(L):
        a_t, b_t = _ab(h, msg_W1[l, :D], msg_W1[l, D:2 * D])
        p0, p1 = _make_edge_kernel(l)(a_t, b_t, c_all, dst2, src2, zn)
        h = _update(h, p0, p1, d0, d1, msg_W2[l], msg_b2[l],
                    upd_W1[l, :D], upd_W1[l, D:], upd_W2[l],
                    upd_b1[l], upd_b2[l])

    return _pool_head(h, batch_f, head_W1, head_b1, head_W2, head_b2)
